# baseline (speedup 1.0000x reference)
import jax
import jax.numpy as jnp
from jax.experimental import pallas as pl

B, HW, DIM = 64, 1024, 64
K = 9 * 114
KPAD = 1152
T = B * HW
TB = 1024


def _vq_body(z_ref, at_ref, a_ref, o_ref):
    zb = z_ref[...]
    nrm = jnp.sqrt(jnp.sum(zb * zb, axis=-1, keepdims=True))
    zn = zb / jnp.maximum(nrm, 1e-12)
    sims = jnp.dot(zn.astype(jnp.bfloat16), at_ref[...],
                   preferred_element_type=jnp.float32)
    col = jax.lax.broadcasted_iota(jnp.int32, (TB, KPAD), 1)
    sims = jnp.where(col < K, sims, -jnp.inf)
    rowmax = jnp.max(sims, axis=-1, keepdims=True)
    first = jnp.min(jnp.where(sims == rowmax, col, KPAD), axis=-1,
                    keepdims=True)
    onehot = (col == first).astype(jnp.float32)
    o_ref[...] = jnp.dot(onehot, a_ref[...],
                         precision=jax.lax.Precision.HIGHEST,
                         preferred_element_type=jnp.float32)


def kernel(z, attractors):
    A = attractors.reshape(-1, DIM)
    a_pad = jnp.zeros((KPAD, DIM), A.dtype).at[:K].set(A)
    at_pad = a_pad.T.astype(jnp.bfloat16)
    zf = z.reshape(T, DIM)
    out = pl.pallas_call(
        _vq_body,
        grid=(T // TB,),
        in_specs=[
            pl.BlockSpec((TB, DIM), lambda i: (i, 0)),
            pl.BlockSpec((DIM, KPAD), lambda i: (0, 0)),
            pl.BlockSpec((KPAD, DIM), lambda i: (0, 0)),
        ],
        out_specs=pl.BlockSpec((TB, DIM), lambda i: (i, 0)),
        out_shape=jax.ShapeDtypeStruct((T, DIM), jnp.float32),
    )(zf, at_pad, a_pad)
    return out.reshape(B, HW, DIM)


# speedup vs baseline: 1.7493x; 1.7493x over previous
import jax
import jax.numpy as jnp
from jax.experimental import pallas as pl

B, HW, DIM = 64, 1024, 64
K = 9 * 114
KPAD = 1152
T = B * HW
TB = 1024


def _vq_body(z_ref, at_ref, a_ref, o_ref):
    zb = z_ref[...]
    nrm = jnp.sqrt(jnp.sum(zb * zb, axis=-1, keepdims=True))
    zn = zb / jnp.maximum(nrm, 1e-12)
    sims = jnp.dot(zn.astype(jnp.bfloat16), at_ref[...],
                   preferred_element_type=jnp.float32)
    idx = jnp.argmax(sims, axis=-1)
    col = jax.lax.broadcasted_iota(jnp.int32, (TB, KPAD), 1)
    onehot = (col == idx[:, None]).astype(jnp.bfloat16)
    o_ref[...] = jnp.dot(onehot, a_ref[...],
                         preferred_element_type=jnp.float32)


def kernel(z, attractors):
    A = attractors.reshape(-1, DIM)
    a_pad = jnp.concatenate(
        [A, jnp.broadcast_to(A[:1], (KPAD - K, DIM))], axis=0)
    at_pad = a_pad.T.astype(jnp.bfloat16)
    a_bf = a_pad.astype(jnp.bfloat16)
    zf = z.reshape(T, DIM)
    out = pl.pallas_call(
        _vq_body,
        grid=(T // TB,),
        in_specs=[
            pl.BlockSpec((TB, DIM), lambda i: (i, 0)),
            pl.BlockSpec((DIM, KPAD), lambda i: (0, 0)),
            pl.BlockSpec((KPAD, DIM), lambda i: (0, 0)),
        ],
        out_specs=pl.BlockSpec((TB, DIM), lambda i: (i, 0)),
        out_shape=jax.ShapeDtypeStruct((T, DIM), jnp.float32),
    )(zf, at_pad, a_bf)
    return out.reshape(B, HW, DIM)
